# baseline (speedup 1.0000x reference)
import functools

import jax
import jax.numpy as jnp
from jax import lax
from jax.experimental import pallas as pl
from jax.experimental.pallas import tpu as pltpu
from jax.experimental.pallas import tpu_sc as plsc

NU = 1000000
NM = 100000
K = 64
B = 16384
NW = 32
BPW = B // NW
NCHUNK = 4
CHUNK = BPW // NCHUNK

_mesh = plsc.VectorSubcoreMesh(core_axis_name="c", subcore_axis_name="s")


@functools.partial(
    pl.kernel,
    out_type=jax.ShapeDtypeStruct((B,), jnp.float32),
    mesh=_mesh,
    compiler_params=pltpu.CompilerParams(needs_layout_passes=False,
                                         use_tc_tiling_on_sc=False),
    scratch_types=[
        pltpu.VMEM((NCHUNK, CHUNK), jnp.int32),
        pltpu.VMEM((NCHUNK, CHUNK), jnp.int32),
        pltpu.VMEM((K, CHUNK), jnp.float32),
        pltpu.VMEM((K, CHUNK), jnp.float32),
        pltpu.VMEM((BPW,), jnp.float32),
        pltpu.VMEM((BPW,), jnp.float32),
        pltpu.VMEM((16,), jnp.float32),
        pltpu.VMEM((BPW,), jnp.float32),
        pltpu.SemaphoreType.DMA,
    ],
)
def _mf_kernel(users_hbm, movies_hbm, ut_hbm, mt_hbm, ub_hbm, mb_hbm,
               gb_hbm, out_hbm,
               uidx_v, midx_v, urows, mrows, ubias_v, mbias_v,
               gbias_v, out_v, sem):
    wid = lax.axis_index("s") * 2 + lax.axis_index("c")

    pltpu.sync_copy(users_hbm.at[wid], uidx_v)
    pltpu.sync_copy(movies_hbm.at[wid], midx_v)
    pltpu.sync_copy(gb_hbm, gbias_v)

    def chunk_body(c, carry):
        coff = pl.multiple_of(c * CHUNK, CHUNK)
        cps = [
            pltpu.async_copy(ub_hbm.at[uidx_v.at[c]],
                             ubias_v.at[pl.ds(coff, CHUNK)], sem),
            pltpu.async_copy(mb_hbm.at[midx_v.at[c]],
                             mbias_v.at[pl.ds(coff, CHUNK)], sem),
        ]
        for k in range(K):
            cps.append(pltpu.async_copy(ut_hbm.at[k].at[uidx_v.at[c]],
                                        urows.at[k], sem))
            cps.append(pltpu.async_copy(mt_hbm.at[k].at[midx_v.at[c]],
                                        mrows.at[k], sem))
        for cp in cps:
            cp.wait()

        gb = gbias_v[...]

        def q_body(q, qcarry):
            off = pl.multiple_of(q * 16, 16)
            acc = (ubias_v[pl.ds(coff + off, 16)]
                   + mbias_v[pl.ds(coff + off, 16)] + gb)
            for k in range(K):
                acc = acc + urows[k, pl.ds(off, 16)] * mrows[k, pl.ds(off, 16)]
            out_v[pl.ds(coff + off, 16)] = jnp.clip(acc, 0.5, 5.0)
            return qcarry

        lax.fori_loop(0, CHUNK // 16, q_body, 0)
        return carry

    lax.fori_loop(0, NCHUNK, chunk_body, 0)

    pltpu.sync_copy(out_v, out_hbm.at[pl.ds(pl.multiple_of(wid * BPW, BPW), BPW)])


def kernel(users, movies, user_emb, movie_emb, user_bias, movie_bias, global_bias):
    users_r = users.astype(jnp.int32).reshape(NW, NCHUNK, CHUNK)
    movies_r = movies.astype(jnp.int32).reshape(NW, NCHUNK, CHUNK)
    gb = jnp.full((16,), global_bias, jnp.float32)
    return _mf_kernel(users_r, movies_r, user_emb.T, movie_emb.T,
                      user_bias.reshape(-1), movie_bias.reshape(-1), gb)


# speedup vs baseline: 10.3526x; 10.3526x over previous
import functools

import jax
import jax.numpy as jnp
from jax import lax
from jax.experimental import pallas as pl
from jax.experimental.pallas import tpu as pltpu
from jax.experimental.pallas import tpu_sc as plsc

NU = 1000000
NM = 100000
K = 64
B = 16384
NW = 32
BPW = B // NW
G = 32
NGRP = BPW // G

_mesh = plsc.VectorSubcoreMesh(core_axis_name="c", subcore_axis_name="s")


@functools.partial(
    pl.kernel,
    out_type=jax.ShapeDtypeStruct((B,), jnp.float32),
    mesh=_mesh,
    compiler_params=pltpu.CompilerParams(needs_layout_passes=False,
                                         use_tc_tiling_on_sc=True),
    scratch_types=[
        pltpu.VMEM((BPW,), jnp.int32),
        pltpu.VMEM((BPW,), jnp.int32),
        pltpu.VMEM((G, 8, K), jnp.float32),
        pltpu.VMEM((G, 8, K), jnp.float32),
        pltpu.VMEM((BPW,), jnp.float32),
        pltpu.VMEM((BPW,), jnp.float32),
        pltpu.VMEM((16,), jnp.float32),
        pltpu.VMEM((BPW,), jnp.float32),
        pltpu.SemaphoreType.DMA,
        pltpu.SemaphoreType.DMA,
    ],
)
def _mf_kernel(users_hbm, movies_hbm, ut_hbm, mt_hbm, ub_hbm, mb_hbm,
               gb_hbm, out_hbm,
               uidx_v, midx_v, ublk, mblk, ubias_v, mbias_v,
               gbias_v, out_v, sem, bsem):
    wid = lax.axis_index("s") * 2 + lax.axis_index("c")
    base = pl.multiple_of(wid * BPW, BPW)

    pltpu.sync_copy(users_hbm.at[pl.ds(base, BPW)], uidx_v)
    pltpu.sync_copy(movies_hbm.at[pl.ds(base, BPW)], midx_v)
    pltpu.sync_copy(gb_hbm, gbias_v)

    bias_cps = []
    for c in range(BPW // 128):
        sl = pl.ds(c * 128, 128)
        bias_cps.append(pltpu.async_copy(ub_hbm.at[uidx_v.at[sl]],
                                         ubias_v.at[sl], bsem))
        bias_cps.append(pltpu.async_copy(mb_hbm.at[midx_v.at[sl]],
                                         mbias_v.at[sl], bsem))
    for cp in bias_cps:
        cp.wait()

    iota16 = lax.iota(jnp.int32, 16)
    gb = gbias_v[...]

    def pick(vec, j):
        return jnp.sum(jnp.where(iota16 == j, vec, 0))

    def group_body(g, carry):
        goff = pl.multiple_of(g * G, G)
        uvecs = [uidx_v[pl.ds(goff + v * 16, 16)] for v in range(G // 16)]
        mvecs = [midx_v[pl.ds(goff + v * 16, 16)] for v in range(G // 16)]
        rus = [pick(uvecs[j // 16], j % 16) for j in range(G)]
        rms = [pick(mvecs[j // 16], j % 16) for j in range(G)]
        cps = []
        for j in range(G):
            ru8 = pl.multiple_of((rus[j] >> 3) * 8, 8)
            rm8 = pl.multiple_of((rms[j] >> 3) * 8, 8)
            cps.append(pltpu.async_copy(ut_hbm.at[pl.ds(ru8, 8)],
                                        ublk.at[j], sem))
            cps.append(pltpu.async_copy(mt_hbm.at[pl.ds(rm8, 8)],
                                        mblk.at[j], sem))
        for cp in cps:
            cp.wait()

        for v in range(G // 16):
            dots = jnp.zeros((16,), jnp.float32)
            for jj in range(16):
                j = v * 16 + jj
                su = jnp.full((16,), rus[j] & 7, jnp.int32)
                sm = jnp.full((16,), rms[j] & 7, jnp.int32)
                s = jnp.zeros((16,), jnp.float32)
                for q in range(K // 16):
                    cols = q * 16 + iota16
                    uv = plsc.load_gather(ublk.at[j], [su, cols])
                    mv = plsc.load_gather(mblk.at[j], [sm, cols])
                    s = s + uv * mv
                dots = jnp.where(iota16 == jj, jnp.sum(s), dots)
            voff = pl.multiple_of(goff + v * 16, 16)
            res = dots + ubias_v[pl.ds(voff, 16)] + mbias_v[pl.ds(voff, 16)] + gb
            out_v[pl.ds(voff, 16)] = jnp.clip(res, 0.5, 5.0)
        return carry

    lax.fori_loop(0, NGRP, group_body, 0)

    pltpu.sync_copy(out_v, out_hbm.at[pl.ds(base, BPW)])


def kernel(users, movies, user_emb, movie_emb, user_bias, movie_bias, global_bias):
    gbv = jnp.full((16,), global_bias, jnp.float32)
    return _mf_kernel(users.astype(jnp.int32), movies.astype(jnp.int32),
                      user_emb, movie_emb,
                      user_bias.reshape(-1), movie_bias.reshape(-1), gbv)


# speedup vs baseline: 10.3899x; 1.0036x over previous
import functools

import jax
import jax.numpy as jnp
from jax import lax
from jax.experimental import pallas as pl
from jax.experimental.pallas import tpu as pltpu
from jax.experimental.pallas import tpu_sc as plsc

NU = 1000000
NM = 100000
K = 64
B = 16384
NW = 32
BPW = B // NW
G = 16
NGRP = BPW // G

_mesh = plsc.VectorSubcoreMesh(core_axis_name="c", subcore_axis_name="s")


@functools.partial(
    pl.kernel,
    out_type=jax.ShapeDtypeStruct((B,), jnp.float32),
    mesh=_mesh,
    compiler_params=pltpu.CompilerParams(needs_layout_passes=False,
                                         use_tc_tiling_on_sc=True),
    scratch_types=[
        pltpu.VMEM((BPW,), jnp.int32),
        pltpu.VMEM((BPW,), jnp.int32),
        pltpu.VMEM((2 * G, 8, K), jnp.float32),
        pltpu.VMEM((2 * G, 8, K), jnp.float32),
        pltpu.VMEM((BPW,), jnp.float32),
        pltpu.VMEM((BPW,), jnp.float32),
        pltpu.VMEM((16,), jnp.float32),
        pltpu.VMEM((BPW,), jnp.float32),
        pltpu.SemaphoreType.DMA,
        pltpu.SemaphoreType.DMA,
        pltpu.SemaphoreType.DMA,
    ],
)
def _mf_kernel(users_hbm, movies_hbm, ut_hbm, mt_hbm, ub_hbm, mb_hbm,
               gb_hbm, out_hbm,
               uidx_v, midx_v, ublk, mblk, ubias_v, mbias_v,
               gbias_v, out_v, sem0, sem1, bsem):
    wid = lax.axis_index("s") * 2 + lax.axis_index("c")
    base = pl.multiple_of(wid * BPW, BPW)

    pltpu.sync_copy(users_hbm.at[pl.ds(base, BPW)], uidx_v)
    pltpu.sync_copy(movies_hbm.at[pl.ds(base, BPW)], midx_v)
    pltpu.sync_copy(gb_hbm, gbias_v)

    bias_cps = []
    for c in range(BPW // 128):
        sl = pl.ds(c * 128, 128)
        bias_cps.append(pltpu.async_copy(ub_hbm.at[uidx_v.at[sl]],
                                         ubias_v.at[sl], bsem))
        bias_cps.append(pltpu.async_copy(mb_hbm.at[midx_v.at[sl]],
                                         mbias_v.at[sl], bsem))
    for cp in bias_cps:
        cp.wait()

    iota16 = lax.iota(jnp.int32, 16)
    gb = gbias_v[...]

    def pick(vec, j):
        return jnp.sum(jnp.where(iota16 == j, vec, 0))

    def load_group(g):
        gv = pl.multiple_of(g * G, G)
        uvec = uidx_v[pl.ds(gv, 16)]
        mvec = midx_v[pl.ds(gv, 16)]
        rus = [pick(uvec, j) for j in range(G)]
        rms = [pick(mvec, j) for j in range(G)]
        return rus, rms

    def fire(rus, rms, sbase, sem):
        cps = []
        for j in range(G):
            lu = pl.multiple_of((rus[j] >> 3) * 8, 8)
            lm = pl.multiple_of((rms[j] >> 3) * 8, 8)
            cps.append(pltpu.async_copy(ut_hbm.at[pl.ds(lu, 8)],
                                        ublk.at[sbase + j], sem))
            cps.append(pltpu.async_copy(mt_hbm.at[pl.ds(lm, 8)],
                                        mblk.at[sbase + j], sem))
        return cps

    def compute(g, rus, rms, sbase):
        goff = pl.multiple_of(g * G, G)
        dots = jnp.zeros((16,), jnp.float32)
        for j in range(G):
            su = jnp.full((16,), rus[j] & 7, jnp.int32)
            sm = jnp.full((16,), rms[j] & 7, jnp.int32)
            s = jnp.zeros((16,), jnp.float32)
            for q in range(K // 16):
                cols = q * 16 + iota16
                uv = plsc.load_gather(ublk.at[sbase + j], [su, cols])
                mv = plsc.load_gather(mblk.at[sbase + j], [sm, cols])
                s = s + uv * mv
            dots = jnp.where(iota16 == j, jnp.sum(s), dots)
        res = dots + ubias_v[pl.ds(goff, 16)] + mbias_v[pl.ds(goff, 16)] + gb
        out_v[pl.ds(goff, 16)] = jnp.clip(res, 0.5, 5.0)

    r0 = load_group(0)
    fire(*r0, 0, sem0)

    def pair_body(p, carry):
        g0 = pl.multiple_of(p * 2, 2)
        r1 = load_group(g0 + 1)
        cps1 = fire(*r1, G, sem1)
        rE = load_group(g0)
        for _ in range(2 * G):
            pltpu.make_async_copy(ut_hbm.at[pl.ds(0, 8)], ublk.at[0], sem0).wait()
        compute(g0, *rE, 0)
        @pl.when(g0 + 2 < NGRP)
        def _():
            rNE = load_group(g0 + 2)
            fire(*rNE, 0, sem0)
        for _ in range(2 * G):
            pltpu.make_async_copy(mt_hbm.at[pl.ds(0, 8)], mblk.at[G], sem1).wait()
        compute(g0 + 1, *r1, G)
        return carry

    lax.fori_loop(0, NGRP // 2, pair_body, 0)

    pltpu.sync_copy(out_v, out_hbm.at[pl.ds(base, BPW)])


def kernel(users, movies, user_emb, movie_emb, user_bias, movie_bias, global_bias):
    gbv = jnp.full((16,), global_bias, jnp.float32)
    return _mf_kernel(users.astype(jnp.int32), movies.astype(jnp.int32),
                      user_emb, movie_emb,
                      user_bias.reshape(-1), movie_bias.reshape(-1), gbv)
